# baseline (speedup 1.0000x reference)
import functools

import jax
import jax.numpy as jnp
from jax import lax
from jax.experimental import pallas as pl
from jax.experimental.pallas import tpu as pltpu
from jax.experimental.pallas import tpu_sc as plsc

B, S = 2, 2048
H = 2048
NUM_CB = 32
AV3 = 2054
P = B * S
P_SC = 2560
NP_TC = P - P_SC
NW = 32
PPW = P_SC // NW
BLK = 8
NBLK = PPW // BLK
IDS_PAD = 64
LANES = 16
HBLK = 256
NHB = H // HBLK
NACC = HBLK // LANES


_mesh = plsc.VectorSubcoreMesh(core_axis_name="c", subcore_axis_name="s")


@functools.partial(
    pl.kernel,
    out_type=jax.ShapeDtypeStruct((P_SC, H), jnp.float32),
    mesh=_mesh,
    compiler_params=pltpu.CompilerParams(needs_layout_passes=False),
    scratch_types=[
        pltpu.VMEM((PPW * IDS_PAD,), jnp.int32),
        pltpu.VMEM((2, LANES), jnp.int32),
        pltpu.VMEM((NUM_CB,), jnp.float32),
        pltpu.VMEM((LANES,), jnp.int32),
        pltpu.VMEM((LANES,), jnp.float32),
        pltpu.VMEM((2, LANES, H), jnp.float32),
        pltpu.VMEM((LANES, H), jnp.float32),
        pltpu.VMEM((BLK, H), jnp.float32),
        pltpu.SemaphoreType.DMA,
        pltpu.SemaphoreType.DMA,
        pltpu.SemaphoreType.DMA,
    ],
)
def _embed_kernel(ids_hbm, text_hbm, audio_hbm, out_hbm,
                  tok_v, aidx_v, amask_v, tidx_v, tmask_v,
                  arows_v, trows_v, acc_v, sem0, sem1, sem_t):
    wid = lax.axis_index("s") * 2 + lax.axis_index("c")
    base = wid * PPW
    lanes = lax.broadcasted_iota(jnp.int32, (LANES,), 0)
    sems = (sem0, sem1)

    pltpu.sync_copy(ids_hbm.at[pl.ds(base * IDS_PAD, PPW * IDS_PAD)], tok_v)

    def splat(ref, i):
        return plsc.load_gather(ref, [jnp.full((LANES,), i, jnp.int32)])

    def start_half(gp, half, slot):
        atok = tok_v[pl.ds(gp * IDS_PAD + half * LANES, LANES)]
        aidx_v[slot, :] = atok + (lanes + half * LANES) * AV3
        pltpu.async_copy(
            audio_hbm.at[aidx_v.at[slot]], arows_v.at[slot], sems[slot])

    def wait_slot(slot):
        pltpu.make_async_copy(
            audio_hbm.at[aidx_v.at[slot]], arows_v.at[slot],
            sems[slot]).wait()

    def pair_body(bp, _):
        pair0 = bp * (2 * BLK)

        rowsel = pair0 + lanes
        ttok = plsc.load_gather(tok_v, [rowsel * IDS_PAD + NUM_CB])
        tidx_v[...] = ttok
        tmask_v[...] = jnp.where(ttok != 0, 1.0, 0.0)
        tcopy = pltpu.async_copy(text_hbm.at[tidx_v], trows_v, sem_t)

        start_half(pair0, 0, 0)
        tcopy.wait()

        for sub in (0, 1):
            sub0 = pair0 + sub * BLK

            def pos_body(p, _):
                pbase = (sub0 + p) * IDS_PAD
                atok_lo = tok_v[pl.ds(pbase, LANES)]
                atok_hi = tok_v[pl.ds(pbase + LANES, LANES)]
                amask_v[pl.ds(0, LANES)] = jnp.where(atok_lo != 0, 1.0, 0.0)
                amask_v[pl.ds(LANES, LANES)] = jnp.where(atok_hi != 0, 1.0, 0.0)
                tscale = splat(tmask_v, sub * BLK + p)

                for half in (0, 1):
                    slot = half
                    wait_slot(slot)
                    if half == 0:
                        start_half(sub0 + p, 1, 1)
                    elif sub == 0:
                        start_half(sub0 + p + 1, 0, 0)
                    else:
                        @pl.when(p + 1 < BLK)
                        def _():
                            start_half(sub0 + p + 1, 0, 0)

                    rows = arows_v.at[slot]

                    def hb_body(hb, _):
                        hoff = hb * HBLK
                        if half == 0:
                            accs = [
                                trows_v[sub * BLK + p,
                                        pl.ds(hoff + k * LANES, LANES)] * tscale
                                for k in range(NACC)
                            ]
                        else:
                            accs = [
                                acc_v[p, pl.ds(hoff + k * LANES, LANES)]
                                for k in range(NACC)
                            ]

                        def r_body(r, accs):
                            scale = splat(amask_v, half * LANES + r)
                            return [
                                a + rows[r, pl.ds(hoff + k * LANES, LANES)]
                                * scale
                                for k, a in enumerate(accs)
                            ]

                        accs = lax.fori_loop(0, LANES, r_body, accs)
                        for k in range(NACC):
                            acc_v[p, pl.ds(hoff + k * LANES, LANES)] = accs[k]
                        return 0

                    lax.fori_loop(0, NHB, hb_body, 0)
                return 0

            lax.fori_loop(0, BLK, pos_body, 0)
            pltpu.sync_copy(acc_v, out_hbm.at[pl.ds(base + sub0, BLK), :])
        return 0

    lax.fori_loop(0, NBLK // 2, pair_body, 0)


TCK = NUM_CB + 1
PB = 8
NSTEP = NP_TC // PB
ARS = PB * NUM_CB
RPS = ARS + PB


def _tc_issue(tidx_ref, aidx_ref, text_hbm, audio_hbm, rows_v, sem, step):
    def a_body(r, _):
        idx = aidx_ref[step * ARS + r]
        pltpu.make_async_copy(
            audio_hbm.at[pl.ds(idx, 1), :],
            rows_v.at[pl.ds(r, 1), :], sem).start()
        return 0

    lax.fori_loop(0, ARS, a_body, 0)

    def t_body(p, _):
        idx = tidx_ref[step * PB + p]
        pltpu.make_async_copy(
            text_hbm.at[pl.ds(idx, 1), :],
            rows_v.at[pl.ds(ARS + p, 1), :], sem).start()
        return 0

    lax.fori_loop(0, PB, t_body, 0)


def _tc_kernel(tidx_ref, aidx_ref, text_hbm, audio_hbm, out_ref,
               rows_v, sem0, sem1):
    step = pl.program_id(0)
    parity = step % 2

    @pl.when(step == 0)
    def _():
        _tc_issue(tidx_ref, aidx_ref, text_hbm, audio_hbm,
                  rows_v.at[0], sem0, step)

    @pl.when((parity == 0) & (step + 1 < NSTEP))
    def _():
        _tc_issue(tidx_ref, aidx_ref, text_hbm, audio_hbm,
                  rows_v.at[1], sem1, step + 1)

    @pl.when((parity == 1) & (step + 1 < NSTEP))
    def _():
        _tc_issue(tidx_ref, aidx_ref, text_hbm, audio_hbm,
                  rows_v.at[0], sem0, step + 1)

    @pl.when(parity == 0)
    def _():
        pltpu.make_async_copy(
            audio_hbm.at[pl.ds(0, RPS), :], rows_v.at[0], sem0).wait()

    @pl.when(parity == 1)
    def _():
        pltpu.make_async_copy(
            audio_hbm.at[pl.ds(0, RPS), :], rows_v.at[1], sem1).wait()

    for p in range(PB):
        acc = None
        for g in range(NUM_CB // 8):
            rows8 = rows_v[parity, pl.ds(p * NUM_CB + g * 8, 8), :]
            scales = jnp.stack([
                jnp.where(
                    aidx_ref[step * ARS + p * NUM_CB + g * 8 + k]
                    != (g * 8 + k) * AV3, 1.0, 0.0).astype(jnp.float32)
                for k in range(8)
            ]).reshape(8, 1)
            term = rows8 * scales
            acc = term if acc is None else acc + term
        trow = rows_v[parity, pl.ds(ARS + p, 1), :]
        tscale = jnp.where(
            tidx_ref[step * PB + p] != 0, 1.0, 0.0).astype(jnp.float32)
        result = jnp.sum(acc, axis=0, keepdims=True) + trow * tscale
        out_ref[pl.ds(p, 1), :] = result


_tc_call = pl.pallas_call(
    _tc_kernel,
    grid_spec=pltpu.PrefetchScalarGridSpec(
        num_scalar_prefetch=2,
        grid=(NSTEP,),
        in_specs=[
            pl.BlockSpec(memory_space=pl.ANY),
            pl.BlockSpec(memory_space=pl.ANY),
        ],
        out_specs=pl.BlockSpec((PB, H), lambda i, t, a: (i, 0)),
        scratch_shapes=[
            pltpu.VMEM((2, RPS, H), jnp.float32),
            pltpu.SemaphoreType.DMA,
            pltpu.SemaphoreType.DMA,
        ],
    ),
    out_shape=jax.ShapeDtypeStruct((NP_TC, H), jnp.float32),
)


def kernel(input_ids, text_table, audio_table):
    ids = input_ids.reshape(P, NUM_CB + 1).astype(jnp.int32)
    ids_pad = (
        jnp.pad(ids[:P_SC], ((0, 0), (0, IDS_PAD - (NUM_CB + 1))))
        .reshape(-1))
    sc_out = _embed_kernel(ids_pad, text_table, audio_table)

    tc_ids = ids[P_SC:]
    offs = jnp.arange(NUM_CB, dtype=jnp.int32) * AV3
    tc_aidx = (tc_ids[:, :-1] + offs[None, :]).reshape(-1)
    tc_tidx = tc_ids[:, -1].reshape(-1)
    tc_out = _tc_call(tc_tidx, tc_aidx, text_table, audio_table)

    return jnp.concatenate([sc_out, tc_out], axis=0).reshape(B, S, H)


# speedup vs baseline: 1.3730x; 1.3730x over previous
import functools

import jax
import jax.numpy as jnp
from jax import lax
from jax.experimental import pallas as pl
from jax.experimental.pallas import tpu as pltpu
from jax.experimental.pallas import tpu_sc as plsc

B, S = 2, 2048
H = 2048
NUM_CB = 32
AV3 = 2054
P = B * S
P_SC = 3072
NP_TC = P - P_SC
NW = 32
PPW = P_SC // NW
BLK = 8
NBLK = PPW // BLK
IDS_PAD = 64
LANES = 16
HBLK = 256
NHB = H // HBLK
NACC = HBLK // LANES


_mesh = plsc.VectorSubcoreMesh(core_axis_name="c", subcore_axis_name="s")


@functools.partial(
    pl.kernel,
    out_type=jax.ShapeDtypeStruct((P_SC, H), jnp.float32),
    mesh=_mesh,
    compiler_params=pltpu.CompilerParams(needs_layout_passes=False),
    scratch_types=[
        pltpu.VMEM((PPW * IDS_PAD,), jnp.int32),
        pltpu.VMEM((2, LANES), jnp.int32),
        pltpu.VMEM((NUM_CB,), jnp.float32),
        pltpu.VMEM((LANES,), jnp.int32),
        pltpu.VMEM((LANES,), jnp.float32),
        pltpu.VMEM((2, LANES, H), jnp.float32),
        pltpu.VMEM((LANES, H), jnp.float32),
        pltpu.VMEM((BLK, H), jnp.float32),
        pltpu.SemaphoreType.DMA,
        pltpu.SemaphoreType.DMA,
        pltpu.SemaphoreType.DMA,
    ],
)
def _embed_kernel(ids_hbm, text_hbm, audio_hbm, out_hbm,
                  tok_v, aidx_v, amask_v, tidx_v, tmask_v,
                  arows_v, trows_v, acc_v, sem0, sem1, sem_t):
    wid = lax.axis_index("s") * 2 + lax.axis_index("c")
    base = wid * PPW
    lanes = lax.broadcasted_iota(jnp.int32, (LANES,), 0)
    sems = (sem0, sem1)

    pltpu.sync_copy(ids_hbm.at[pl.ds(base * IDS_PAD, PPW * IDS_PAD)], tok_v)

    def splat(ref, i):
        return plsc.load_gather(ref, [jnp.full((LANES,), i, jnp.int32)])

    def start_half(gp, half, slot):
        atok = tok_v[pl.ds(gp * IDS_PAD + half * LANES, LANES)]
        aidx_v[slot, :] = atok + (lanes + half * LANES) * AV3
        pltpu.async_copy(
            audio_hbm.at[aidx_v.at[slot]], arows_v.at[slot], sems[slot])

    def wait_slot(slot):
        pltpu.make_async_copy(
            audio_hbm.at[aidx_v.at[slot]], arows_v.at[slot],
            sems[slot]).wait()

    def pair_body(bp, _):
        pair0 = bp * (2 * BLK)

        rowsel = pair0 + lanes
        ttok = plsc.load_gather(tok_v, [rowsel * IDS_PAD + NUM_CB])
        tidx_v[...] = ttok
        tmask_v[...] = jnp.where(ttok != 0, 1.0, 0.0)
        tcopy = pltpu.async_copy(text_hbm.at[tidx_v], trows_v, sem_t)

        start_half(pair0, 0, 0)
        tcopy.wait()

        for sub in (0, 1):
            sub0 = pair0 + sub * BLK

            def pos_body(p, _):
                pbase = (sub0 + p) * IDS_PAD
                atok_lo = tok_v[pl.ds(pbase, LANES)]
                atok_hi = tok_v[pl.ds(pbase + LANES, LANES)]
                amask_v[pl.ds(0, LANES)] = jnp.where(atok_lo != 0, 1.0, 0.0)
                amask_v[pl.ds(LANES, LANES)] = jnp.where(atok_hi != 0, 1.0, 0.0)
                tscale = splat(tmask_v, sub * BLK + p)

                for half in (0, 1):
                    slot = half
                    wait_slot(slot)
                    if half == 0:
                        start_half(sub0 + p, 1, 1)
                    elif sub == 0:
                        start_half(sub0 + p + 1, 0, 0)
                    else:
                        @pl.when(p + 1 < BLK)
                        def _():
                            start_half(sub0 + p + 1, 0, 0)

                    rows = arows_v.at[slot]

                    def hb_body(hb, _):
                        hoff = hb * HBLK
                        if half == 0:
                            accs = [
                                trows_v[sub * BLK + p,
                                        pl.ds(hoff + k * LANES, LANES)] * tscale
                                for k in range(NACC)
                            ]
                        else:
                            accs = [
                                acc_v[p, pl.ds(hoff + k * LANES, LANES)]
                                for k in range(NACC)
                            ]

                        def r_body(r, accs):
                            scale = splat(amask_v, half * LANES + r)
                            return [
                                a + rows[r, pl.ds(hoff + k * LANES, LANES)]
                                * scale
                                for k, a in enumerate(accs)
                            ]

                        accs = lax.fori_loop(0, LANES, r_body, accs)
                        for k in range(NACC):
                            acc_v[p, pl.ds(hoff + k * LANES, LANES)] = accs[k]
                        return 0

                    lax.fori_loop(0, NHB, hb_body, 0)
                return 0

            lax.fori_loop(0, BLK, pos_body, 0)
            pltpu.sync_copy(acc_v, out_hbm.at[pl.ds(base + sub0, BLK), :])
        return 0

    lax.fori_loop(0, NBLK // 2, pair_body, 0)


TCK = NUM_CB + 1
PB = 16
NSTEP = NP_TC // PB
ARS = PB * NUM_CB
RPS = ARS + PB


def _tc_issue(tidx_ref, aidx_ref, text_hbm, audio_hbm, rows_v, sem, step):
    def a_body(r, _):
        idx = aidx_ref[step * ARS + r]
        pltpu.make_async_copy(
            audio_hbm.at[pl.ds(idx, 1), :],
            rows_v.at[pl.ds(r, 1), :], sem).start()
        return 0

    lax.fori_loop(0, ARS, a_body, 0, unroll=8)

    def t_body(p, _):
        idx = tidx_ref[step * PB + p]
        pltpu.make_async_copy(
            text_hbm.at[pl.ds(idx, 1), :],
            rows_v.at[pl.ds(ARS + p, 1), :], sem).start()
        return 0

    lax.fori_loop(0, PB, t_body, 0, unroll=8)


def _tc_kernel(tidx_ref, aidx_ref, text_hbm, audio_hbm, out_ref,
               rows_v, sem0, sem1):
    step = pl.program_id(0)
    parity = step % 2

    @pl.when(step == 0)
    def _():
        _tc_issue(tidx_ref, aidx_ref, text_hbm, audio_hbm,
                  rows_v.at[0], sem0, step)

    @pl.when((parity == 0) & (step + 1 < NSTEP))
    def _():
        _tc_issue(tidx_ref, aidx_ref, text_hbm, audio_hbm,
                  rows_v.at[1], sem1, step + 1)

    @pl.when((parity == 1) & (step + 1 < NSTEP))
    def _():
        _tc_issue(tidx_ref, aidx_ref, text_hbm, audio_hbm,
                  rows_v.at[0], sem0, step + 1)

    @pl.when(parity == 0)
    def _():
        pltpu.make_async_copy(
            audio_hbm.at[pl.ds(0, RPS), :], rows_v.at[0], sem0).wait()

    @pl.when(parity == 1)
    def _():
        pltpu.make_async_copy(
            audio_hbm.at[pl.ds(0, RPS), :], rows_v.at[1], sem1).wait()

    for p in range(PB):
        acc = None
        for g in range(NUM_CB // 8):
            rows8 = rows_v[parity, pl.ds(p * NUM_CB + g * 8, 8), :]
            scales = jnp.stack([
                jnp.where(
                    aidx_ref[step * ARS + p * NUM_CB + g * 8 + k]
                    != (g * 8 + k) * AV3, 1.0, 0.0).astype(jnp.float32)
                for k in range(8)
            ]).reshape(8, 1)
            term = rows8 * scales
            acc = term if acc is None else acc + term
        trow = rows_v[parity, pl.ds(ARS + p, 1), :]
        tscale = jnp.where(
            tidx_ref[step * PB + p] != 0, 1.0, 0.0).astype(jnp.float32)
        result = jnp.sum(acc, axis=0, keepdims=True) + trow * tscale
        out_ref[pl.ds(p, 1), :] = result


_tc_call = pl.pallas_call(
    _tc_kernel,
    grid_spec=pltpu.PrefetchScalarGridSpec(
        num_scalar_prefetch=2,
        grid=(NSTEP,),
        in_specs=[
            pl.BlockSpec(memory_space=pl.ANY),
            pl.BlockSpec(memory_space=pl.ANY),
        ],
        out_specs=pl.BlockSpec((PB, H), lambda i, t, a: (i, 0)),
        scratch_shapes=[
            pltpu.VMEM((2, RPS, H), jnp.float32),
            pltpu.SemaphoreType.DMA,
            pltpu.SemaphoreType.DMA,
        ],
    ),
    out_shape=jax.ShapeDtypeStruct((NP_TC, H), jnp.float32),
)


def kernel(input_ids, text_table, audio_table):
    ids = input_ids.reshape(P, NUM_CB + 1).astype(jnp.int32)
    ids_pad = (
        jnp.pad(ids[:P_SC], ((0, 0), (0, IDS_PAD - (NUM_CB + 1))))
        .reshape(-1))
    sc_out = _embed_kernel(ids_pad, text_table, audio_table)

    tc_ids = ids[P_SC:]
    offs = jnp.arange(NUM_CB, dtype=jnp.int32) * AV3
    tc_aidx = (tc_ids[:, :-1] + offs[None, :]).reshape(-1)
    tc_tidx = tc_ids[:, -1].reshape(-1)
    tc_out = _tc_call(tc_tidx, tc_aidx, text_table, audio_table)

    return jnp.concatenate([sc_out, tc_out], axis=0).reshape(B, S, H)


# speedup vs baseline: 1.3875x; 1.0106x over previous
import functools

import jax
import jax.numpy as jnp
from jax import lax
from jax.experimental import pallas as pl
from jax.experimental.pallas import tpu as pltpu
from jax.experimental.pallas import tpu_sc as plsc

B, S = 2, 2048
H = 2048
NUM_CB = 32
AV3 = 2054
P = B * S
P_SC = 3072
NP_TC = P - P_SC
NW = 32
PPW = P_SC // NW
BLK = 8
NBLK = PPW // BLK
IDS_PAD = 64
LANES = 16
HBLK = 256
NHB = H // HBLK
NACC = HBLK // LANES


_mesh = plsc.VectorSubcoreMesh(core_axis_name="c", subcore_axis_name="s")


@functools.partial(
    pl.kernel,
    out_type=jax.ShapeDtypeStruct((P_SC, H), jnp.float32),
    mesh=_mesh,
    compiler_params=pltpu.CompilerParams(needs_layout_passes=False),
    scratch_types=[
        pltpu.VMEM((PPW * IDS_PAD,), jnp.int32),
        pltpu.VMEM((2, LANES), jnp.int32),
        pltpu.VMEM((NUM_CB,), jnp.float32),
        pltpu.VMEM((LANES,), jnp.int32),
        pltpu.VMEM((LANES,), jnp.float32),
        pltpu.VMEM((2, LANES, H), jnp.float32),
        pltpu.VMEM((LANES, H), jnp.float32),
        pltpu.VMEM((BLK, H), jnp.float32),
        pltpu.SemaphoreType.DMA,
        pltpu.SemaphoreType.DMA,
        pltpu.SemaphoreType.DMA,
    ],
)
def _embed_kernel(ids_hbm, text_hbm, audio_hbm, out_hbm,
                  tok_v, aidx_v, amask_v, tidx_v, tmask_v,
                  arows_v, trows_v, acc_v, sem0, sem1, sem_t):
    wid = lax.axis_index("s") * 2 + lax.axis_index("c")
    base = wid * PPW
    lanes = lax.broadcasted_iota(jnp.int32, (LANES,), 0)
    sems = (sem0, sem1)

    pltpu.sync_copy(ids_hbm.at[pl.ds(base * IDS_PAD, PPW * IDS_PAD)], tok_v)

    def splat(ref, i):
        return plsc.load_gather(ref, [jnp.full((LANES,), i, jnp.int32)])

    def start_half(gp, half, slot):
        atok = tok_v[pl.ds(gp * IDS_PAD + half * LANES, LANES)]
        aidx_v[slot, :] = atok + (lanes + half * LANES) * AV3
        pltpu.async_copy(
            audio_hbm.at[aidx_v.at[slot]], arows_v.at[slot], sems[slot])

    def wait_slot(slot):
        pltpu.make_async_copy(
            audio_hbm.at[aidx_v.at[slot]], arows_v.at[slot],
            sems[slot]).wait()

    def pair_body(bp, _):
        pair0 = bp * (2 * BLK)

        rowsel = pair0 + lanes
        ttok = plsc.load_gather(tok_v, [rowsel * IDS_PAD + NUM_CB])
        tidx_v[...] = ttok
        tmask_v[...] = jnp.where(ttok != 0, 1.0, 0.0)
        tcopy = pltpu.async_copy(text_hbm.at[tidx_v], trows_v, sem_t)

        start_half(pair0, 0, 0)
        tcopy.wait()

        for sub in (0, 1):
            sub0 = pair0 + sub * BLK

            def pos_body(p, _):
                pbase = (sub0 + p) * IDS_PAD
                atok_lo = tok_v[pl.ds(pbase, LANES)]
                atok_hi = tok_v[pl.ds(pbase + LANES, LANES)]
                amask_v[pl.ds(0, LANES)] = jnp.where(atok_lo != 0, 1.0, 0.0)
                amask_v[pl.ds(LANES, LANES)] = jnp.where(atok_hi != 0, 1.0, 0.0)
                tscale = splat(tmask_v, sub * BLK + p)

                for half in (0, 1):
                    slot = half
                    wait_slot(slot)
                    if half == 0:
                        start_half(sub0 + p, 1, 1)
                    elif sub == 0:
                        start_half(sub0 + p + 1, 0, 0)
                    else:
                        @pl.when(p + 1 < BLK)
                        def _():
                            start_half(sub0 + p + 1, 0, 0)

                    rows = arows_v.at[slot]

                    def hb_body(hb, _):
                        hoff = hb * HBLK
                        if half == 0:
                            accs = [
                                trows_v[sub * BLK + p,
                                        pl.ds(hoff + k * LANES, LANES)] * tscale
                                for k in range(NACC)
                            ]
                        else:
                            accs = [
                                acc_v[p, pl.ds(hoff + k * LANES, LANES)]
                                for k in range(NACC)
                            ]

                        def r_body(r, accs):
                            scale = splat(amask_v, half * LANES + r)
                            return [
                                a + rows[r, pl.ds(hoff + k * LANES, LANES)]
                                * scale
                                for k, a in enumerate(accs)
                            ]

                        accs = lax.fori_loop(0, LANES, r_body, accs)
                        for k in range(NACC):
                            acc_v[p, pl.ds(hoff + k * LANES, LANES)] = accs[k]
                        return 0

                    lax.fori_loop(0, NHB, hb_body, 0)
                return 0

            lax.fori_loop(0, BLK, pos_body, 0)
            pltpu.sync_copy(acc_v, out_hbm.at[pl.ds(base + sub0, BLK), :])
        return 0

    lax.fori_loop(0, NBLK // 2, pair_body, 0)


TCK = NUM_CB + 1
PB = 8
NSTEP = NP_TC // PB
ARS = PB * NUM_CB
RPS = ARS + PB


def _tc_issue(tidx_ref, aidx_ref, text_hbm, audio_hbm, rows_v, sem, step):
    def a_body(r, _):
        idx = aidx_ref[step * ARS + r]
        pltpu.make_async_copy(
            audio_hbm.at[pl.ds(idx, 1), :],
            rows_v.at[pl.ds(r, 1), :], sem).start()
        return 0

    lax.fori_loop(0, ARS, a_body, 0, unroll=8)

    def t_body(p, _):
        idx = tidx_ref[step * PB + p]
        pltpu.make_async_copy(
            text_hbm.at[pl.ds(idx, 1), :],
            rows_v.at[pl.ds(ARS + p, 1), :], sem).start()
        return 0

    lax.fori_loop(0, PB, t_body, 0, unroll=8)


def _tc_kernel(tidx_ref, aidx_ref, text_hbm, audio_hbm, out_ref,
               rows_v, sem0, sem1):
    step = pl.program_id(0)
    parity = step % 2

    @pl.when(step == 0)
    def _():
        _tc_issue(tidx_ref, aidx_ref, text_hbm, audio_hbm,
                  rows_v.at[0], sem0, step)

    @pl.when((parity == 0) & (step + 1 < NSTEP))
    def _():
        _tc_issue(tidx_ref, aidx_ref, text_hbm, audio_hbm,
                  rows_v.at[1], sem1, step + 1)

    @pl.when((parity == 1) & (step + 1 < NSTEP))
    def _():
        _tc_issue(tidx_ref, aidx_ref, text_hbm, audio_hbm,
                  rows_v.at[0], sem0, step + 1)

    @pl.when(parity == 0)
    def _():
        pltpu.make_async_copy(
            audio_hbm.at[pl.ds(0, RPS), :], rows_v.at[0], sem0).wait()

    @pl.when(parity == 1)
    def _():
        pltpu.make_async_copy(
            audio_hbm.at[pl.ds(0, RPS), :], rows_v.at[1], sem1).wait()

    for p in range(PB):
        acc = None
        for g in range(NUM_CB // 8):
            rows8 = rows_v[parity, pl.ds(p * NUM_CB + g * 8, 8), :]
            scales = jnp.stack([
                jnp.where(
                    aidx_ref[step * ARS + p * NUM_CB + g * 8 + k]
                    != (g * 8 + k) * AV3, 1.0, 0.0).astype(jnp.float32)
                for k in range(8)
            ]).reshape(8, 1)
            term = rows8 * scales
            acc = term if acc is None else acc + term
        trow = rows_v[parity, pl.ds(ARS + p, 1), :]
        tscale = jnp.where(
            tidx_ref[step * PB + p] != 0, 1.0, 0.0).astype(jnp.float32)
        result = jnp.sum(acc, axis=0, keepdims=True) + trow * tscale
        out_ref[pl.ds(p, 1), :] = result


_tc_call = pl.pallas_call(
    _tc_kernel,
    grid_spec=pltpu.PrefetchScalarGridSpec(
        num_scalar_prefetch=2,
        grid=(NSTEP,),
        in_specs=[
            pl.BlockSpec(memory_space=pl.ANY),
            pl.BlockSpec(memory_space=pl.ANY),
        ],
        out_specs=pl.BlockSpec((PB, H), lambda i, t, a: (i, 0)),
        scratch_shapes=[
            pltpu.VMEM((2, RPS, H), jnp.float32),
            pltpu.SemaphoreType.DMA,
            pltpu.SemaphoreType.DMA,
        ],
    ),
    out_shape=jax.ShapeDtypeStruct((NP_TC, H), jnp.float32),
)


def kernel(input_ids, text_table, audio_table):
    ids = input_ids.reshape(P, NUM_CB + 1).astype(jnp.int32)
    ids_pad = (
        jnp.pad(ids[:P_SC], ((0, 0), (0, IDS_PAD - (NUM_CB + 1))))
        .reshape(-1))
    sc_out = _embed_kernel(ids_pad, text_table, audio_table)

    tc_ids = ids[P_SC:]
    offs = jnp.arange(NUM_CB, dtype=jnp.int32) * AV3
    tc_aidx = (tc_ids[:, :-1] + offs[None, :]).reshape(-1)
    tc_tidx = tc_ids[:, -1].reshape(-1)
    tc_out = _tc_call(tc_tidx, tc_aidx, text_table, audio_table)

    return jnp.concatenate([sc_out, tc_out], axis=0).reshape(B, S, H)


# speedup vs baseline: 1.4292x; 1.0301x over previous
import functools

import jax
import jax.numpy as jnp
from jax import lax
from jax.experimental import pallas as pl
from jax.experimental.pallas import tpu as pltpu
from jax.experimental.pallas import tpu_sc as plsc

B, S = 2, 2048
H = 2048
NUM_CB = 32
AV3 = 2054
P = B * S
P_SC = 3072
NP_TC = P - P_SC
NW = 32
PPW = P_SC // NW
BLK = 8
NBLK = PPW // BLK
IDS_PAD = 64
LANES = 16
HBLK = 256
NHB = H // HBLK
NACC = HBLK // LANES


_mesh = plsc.VectorSubcoreMesh(core_axis_name="c", subcore_axis_name="s")


@functools.partial(
    pl.kernel,
    out_type=jax.ShapeDtypeStruct((P, H), jnp.float32),
    mesh=_mesh,
    compiler_params=pltpu.CompilerParams(needs_layout_passes=False),
    scratch_types=[
        pltpu.VMEM((PPW * IDS_PAD,), jnp.int32),
        pltpu.VMEM((2, LANES), jnp.int32),
        pltpu.VMEM((NUM_CB,), jnp.float32),
        pltpu.VMEM((LANES,), jnp.int32),
        pltpu.VMEM((LANES,), jnp.float32),
        pltpu.VMEM((2, LANES, H), jnp.float32),
        pltpu.VMEM((LANES, H), jnp.float32),
        pltpu.VMEM((BLK, H), jnp.float32),
        pltpu.SemaphoreType.DMA,
        pltpu.SemaphoreType.DMA,
        pltpu.SemaphoreType.DMA,
    ],
)
def _embed_kernel(ids_hbm, text_hbm, audio_hbm, out_hbm,
                  tok_v, aidx_v, amask_v, tidx_v, tmask_v,
                  arows_v, trows_v, acc_v, sem0, sem1, sem_t):
    wid = lax.axis_index("s") * 2 + lax.axis_index("c")
    base = wid * PPW
    lanes = lax.broadcasted_iota(jnp.int32, (LANES,), 0)
    sems = (sem0, sem1)

    pltpu.sync_copy(ids_hbm.at[pl.ds(base * IDS_PAD, PPW * IDS_PAD)], tok_v)

    def splat(ref, i):
        return plsc.load_gather(ref, [jnp.full((LANES,), i, jnp.int32)])

    def start_half(gp, half, slot):
        atok = tok_v[pl.ds(gp * IDS_PAD + half * LANES, LANES)]
        aidx_v[slot, :] = atok + (lanes + half * LANES) * AV3
        pltpu.async_copy(
            audio_hbm.at[aidx_v.at[slot]], arows_v.at[slot], sems[slot])

    def wait_slot(slot):
        pltpu.make_async_copy(
            audio_hbm.at[aidx_v.at[slot]], arows_v.at[slot],
            sems[slot]).wait()

    def pair_body(bp, _):
        pair0 = bp * (2 * BLK)

        rowsel = pair0 + lanes
        ttok = plsc.load_gather(tok_v, [rowsel * IDS_PAD + NUM_CB])
        tidx_v[...] = ttok
        tmask_v[...] = jnp.where(ttok != 0, 1.0, 0.0)
        tcopy = pltpu.async_copy(text_hbm.at[tidx_v], trows_v, sem_t)

        start_half(pair0, 0, 0)
        tcopy.wait()

        for sub in (0, 1):
            sub0 = pair0 + sub * BLK

            def pos_body(p, _):
                pbase = (sub0 + p) * IDS_PAD
                atok_lo = tok_v[pl.ds(pbase, LANES)]
                atok_hi = tok_v[pl.ds(pbase + LANES, LANES)]
                amask_v[pl.ds(0, LANES)] = jnp.where(atok_lo != 0, 1.0, 0.0)
                amask_v[pl.ds(LANES, LANES)] = jnp.where(atok_hi != 0, 1.0, 0.0)
                tscale = splat(tmask_v, sub * BLK + p)

                for half in (0, 1):
                    slot = half
                    wait_slot(slot)
                    if half == 0:
                        start_half(sub0 + p, 1, 1)
                    elif sub == 0:
                        start_half(sub0 + p + 1, 0, 0)
                    else:
                        @pl.when(p + 1 < BLK)
                        def _():
                            start_half(sub0 + p + 1, 0, 0)

                    rows = arows_v.at[slot]

                    def hb_body(hb, _):
                        hoff = hb * HBLK
                        if half == 0:
                            accs = [
                                trows_v[sub * BLK + p,
                                        pl.ds(hoff + k * LANES, LANES)] * tscale
                                for k in range(NACC)
                            ]
                        else:
                            accs = [
                                acc_v[p, pl.ds(hoff + k * LANES, LANES)]
                                for k in range(NACC)
                            ]

                        def r_body(r, accs):
                            scale = splat(amask_v, half * LANES + r)
                            return [
                                a + rows[r, pl.ds(hoff + k * LANES, LANES)]
                                * scale
                                for k, a in enumerate(accs)
                            ]

                        accs = lax.fori_loop(0, LANES, r_body, accs)
                        for k in range(NACC):
                            acc_v[p, pl.ds(hoff + k * LANES, LANES)] = accs[k]
                        return 0

                    lax.fori_loop(0, NHB, hb_body, 0)
                return 0

            lax.fori_loop(0, BLK, pos_body, 0)
            pltpu.sync_copy(acc_v, out_hbm.at[pl.ds(base + sub0, BLK), :])
        return 0

    lax.fori_loop(0, NBLK // 2, pair_body, 0)


TCK = NUM_CB + 1
PB = 8
NSTEP = NP_TC // PB
ARS = PB * NUM_CB
RPS = ARS + PB


def _tc_issue(tidx_ref, aidx_ref, text_hbm, audio_hbm, rows_v, sem, step):
    def a_body(r, _):
        idx = aidx_ref[step * ARS + r]
        pltpu.make_async_copy(
            audio_hbm.at[pl.ds(idx, 1), :],
            rows_v.at[pl.ds(r, 1), :], sem).start()
        return 0

    lax.fori_loop(0, ARS, a_body, 0, unroll=8)

    def t_body(p, _):
        idx = tidx_ref[step * PB + p]
        pltpu.make_async_copy(
            text_hbm.at[pl.ds(idx, 1), :],
            rows_v.at[pl.ds(ARS + p, 1), :], sem).start()
        return 0

    lax.fori_loop(0, PB, t_body, 0, unroll=8)


RING = 3


def _tc_kernel(tidx_ref, aidx_ref, text_hbm, audio_hbm, out_ref,
               rows_v, sem0, sem1, sem2):
    step = pl.program_id(0)
    parity = step % RING
    sems = (sem0, sem1, sem2)

    @pl.when(step == 0)
    def _():
        _tc_issue(tidx_ref, aidx_ref, text_hbm, audio_hbm,
                  rows_v.at[0], sems[0], step)
        _tc_issue(tidx_ref, aidx_ref, text_hbm, audio_hbm,
                  rows_v.at[1], sems[1], step + 1)

    for k in range(RING):
        nslot = (k + 2) % RING

        @pl.when((parity == k) & (step + 2 < NSTEP))
        def _(nslot=nslot):
            _tc_issue(tidx_ref, aidx_ref, text_hbm, audio_hbm,
                      rows_v.at[nslot], sems[nslot], step + 2)

    for k in range(RING):
        @pl.when(parity == k)
        def _(k=k):
            pltpu.make_async_copy(
                audio_hbm.at[pl.ds(0, RPS), :], rows_v.at[k], sems[k]).wait()

    for p in range(PB):
        acc = None
        for g in range(NUM_CB // 8):
            rows8 = rows_v[parity, pl.ds(p * NUM_CB + g * 8, 8), :]
            scales = jnp.stack([
                jnp.where(
                    aidx_ref[step * ARS + p * NUM_CB + g * 8 + k]
                    != (g * 8 + k) * AV3, 1.0, 0.0).astype(jnp.float32)
                for k in range(8)
            ]).reshape(8, 1)
            term = rows8 * scales
            acc = term if acc is None else acc + term
        trow = rows_v[parity, pl.ds(ARS + p, 1), :]
        tscale = jnp.where(
            tidx_ref[step * PB + p] != 0, 1.0, 0.0).astype(jnp.float32)
        result = jnp.sum(acc, axis=0, keepdims=True) + trow * tscale
        out_ref[pl.ds(p, 1), :] = result


_tc_call = pl.pallas_call(
    _tc_kernel,
    grid_spec=pltpu.PrefetchScalarGridSpec(
        num_scalar_prefetch=2,
        grid=(NSTEP,),
        in_specs=[
            pl.BlockSpec(memory_space=pl.ANY),
            pl.BlockSpec(memory_space=pl.ANY),
        ],
        out_specs=pl.BlockSpec((PB, H), lambda i, t, a: (i, 0)),
        scratch_shapes=[
            pltpu.VMEM((RING, RPS, H), jnp.float32),
            pltpu.SemaphoreType.DMA,
            pltpu.SemaphoreType.DMA,
            pltpu.SemaphoreType.DMA,
        ],
    ),
    out_shape=jax.ShapeDtypeStruct((NP_TC, H), jnp.float32),
)


def kernel(input_ids, text_table, audio_table):
    ids = input_ids.reshape(P, NUM_CB + 1).astype(jnp.int32)
    ids_pad = (
        jnp.pad(ids[:P_SC], ((0, 0), (0, IDS_PAD - (NUM_CB + 1))))
        .reshape(-1))
    sc_out = _embed_kernel(ids_pad, text_table, audio_table)

    tc_ids = ids[P_SC:]
    offs = jnp.arange(NUM_CB, dtype=jnp.int32) * AV3
    tc_aidx = (tc_ids[:, :-1] + offs[None, :]).reshape(-1)
    tc_tidx = tc_ids[:, -1].reshape(-1)
    tc_out = _tc_call(tc_tidx, tc_aidx, text_table, audio_table)

    out = lax.dynamic_update_slice(sc_out, tc_out, (P_SC, 0))
    return out.reshape(B, S, H)


# speedup vs baseline: 1.4393x; 1.0070x over previous
import functools

import jax
import jax.numpy as jnp
from jax import lax
from jax.experimental import pallas as pl
from jax.experimental.pallas import tpu as pltpu
from jax.experimental.pallas import tpu_sc as plsc

B, S = 2, 2048
H = 2048
NUM_CB = 32
AV3 = 2054
P = B * S
P_SC = 2560
NP_TC = P - P_SC
NW = 32
PPW = P_SC // NW
BLK = 8
NBLK = PPW // BLK
IDS_PAD = 64
LANES = 16
HBLK = 256
NHB = H // HBLK
NACC = HBLK // LANES


_mesh = plsc.VectorSubcoreMesh(core_axis_name="c", subcore_axis_name="s")


@functools.partial(
    pl.kernel,
    out_type=jax.ShapeDtypeStruct((P, H), jnp.float32),
    mesh=_mesh,
    compiler_params=pltpu.CompilerParams(needs_layout_passes=False),
    scratch_types=[
        pltpu.VMEM((PPW * IDS_PAD,), jnp.int32),
        pltpu.VMEM((2, LANES), jnp.int32),
        pltpu.VMEM((NUM_CB,), jnp.float32),
        pltpu.VMEM((LANES,), jnp.int32),
        pltpu.VMEM((LANES,), jnp.float32),
        pltpu.VMEM((2, LANES, H), jnp.float32),
        pltpu.VMEM((LANES, H), jnp.float32),
        pltpu.VMEM((BLK, H), jnp.float32),
        pltpu.SemaphoreType.DMA,
        pltpu.SemaphoreType.DMA,
        pltpu.SemaphoreType.DMA,
    ],
)
def _embed_kernel(ids_hbm, text_hbm, audio_hbm, out_hbm,
                  tok_v, aidx_v, amask_v, tidx_v, tmask_v,
                  arows_v, trows_v, acc_v, sem0, sem1, sem_t):
    wid = lax.axis_index("s") * 2 + lax.axis_index("c")
    base = wid * PPW
    lanes = lax.broadcasted_iota(jnp.int32, (LANES,), 0)
    sems = (sem0, sem1)

    pltpu.sync_copy(ids_hbm.at[pl.ds(base * IDS_PAD, PPW * IDS_PAD)], tok_v)

    def splat(ref, i):
        return plsc.load_gather(ref, [jnp.full((LANES,), i, jnp.int32)])

    def start_half(gp, half, slot):
        atok = tok_v[pl.ds(gp * IDS_PAD + half * LANES, LANES)]
        aidx_v[slot, :] = atok + (lanes + half * LANES) * AV3
        pltpu.async_copy(
            audio_hbm.at[aidx_v.at[slot]], arows_v.at[slot], sems[slot])

    def wait_slot(slot):
        pltpu.make_async_copy(
            audio_hbm.at[aidx_v.at[slot]], arows_v.at[slot],
            sems[slot]).wait()

    def pair_body(bp, _):
        pair0 = bp * (2 * BLK)

        rowsel = pair0 + lanes
        ttok = plsc.load_gather(tok_v, [rowsel * IDS_PAD + NUM_CB])
        tidx_v[...] = ttok
        tmask_v[...] = jnp.where(ttok != 0, 1.0, 0.0)
        tcopy = pltpu.async_copy(text_hbm.at[tidx_v], trows_v, sem_t)

        start_half(pair0, 0, 0)
        tcopy.wait()

        for sub in (0, 1):
            sub0 = pair0 + sub * BLK

            def pos_body(p, _):
                pbase = (sub0 + p) * IDS_PAD
                atok_lo = tok_v[pl.ds(pbase, LANES)]
                atok_hi = tok_v[pl.ds(pbase + LANES, LANES)]
                amask_v[pl.ds(0, LANES)] = jnp.where(atok_lo != 0, 1.0, 0.0)
                amask_v[pl.ds(LANES, LANES)] = jnp.where(atok_hi != 0, 1.0, 0.0)
                tscale = splat(tmask_v, sub * BLK + p)

                for half in (0, 1):
                    slot = half
                    wait_slot(slot)
                    if half == 0:
                        start_half(sub0 + p, 1, 1)
                    elif sub == 0:
                        start_half(sub0 + p + 1, 0, 0)
                    else:
                        @pl.when(p + 1 < BLK)
                        def _():
                            start_half(sub0 + p + 1, 0, 0)

                    rows = arows_v.at[slot]

                    def hb_body(hb, _):
                        hoff = hb * HBLK
                        if half == 0:
                            accs = [
                                trows_v[sub * BLK + p,
                                        pl.ds(hoff + k * LANES, LANES)] * tscale
                                for k in range(NACC)
                            ]
                        else:
                            accs = [
                                acc_v[p, pl.ds(hoff + k * LANES, LANES)]
                                for k in range(NACC)
                            ]

                        def r_body(r, accs):
                            scale = splat(amask_v, half * LANES + r)
                            return [
                                a + rows[r, pl.ds(hoff + k * LANES, LANES)]
                                * scale
                                for k, a in enumerate(accs)
                            ]

                        accs = lax.fori_loop(0, LANES, r_body, accs)
                        for k in range(NACC):
                            acc_v[p, pl.ds(hoff + k * LANES, LANES)] = accs[k]
                        return 0

                    lax.fori_loop(0, NHB, hb_body, 0)
                return 0

            lax.fori_loop(0, BLK, pos_body, 0)
            pltpu.sync_copy(acc_v, out_hbm.at[pl.ds(base + sub0, BLK), :])
        return 0

    lax.fori_loop(0, NBLK // 2, pair_body, 0)


TCK = NUM_CB + 1
PB = 8
NSTEP = NP_TC // PB
ARS = PB * NUM_CB
RPS = ARS + PB


def _tc_issue(tidx_ref, aidx_ref, text_hbm, audio_hbm, rows_v, sem, step):
    def a_body(r, _):
        idx = aidx_ref[step * ARS + r]
        pltpu.make_async_copy(
            audio_hbm.at[pl.ds(idx, 1), :],
            rows_v.at[pl.ds(r, 1), :], sem).start()
        return 0

    lax.fori_loop(0, ARS, a_body, 0, unroll=8)

    def t_body(p, _):
        idx = tidx_ref[step * PB + p]
        pltpu.make_async_copy(
            text_hbm.at[pl.ds(idx, 1), :],
            rows_v.at[pl.ds(ARS + p, 1), :], sem).start()
        return 0

    lax.fori_loop(0, PB, t_body, 0, unroll=8)


RING = 3


def _tc_kernel(tidx_ref, aidx_ref, text_hbm, audio_hbm, out_ref,
               rows_v, sem0, sem1, sem2):
    step = pl.program_id(0)
    parity = step % RING
    sems = (sem0, sem1, sem2)

    @pl.when(step == 0)
    def _():
        _tc_issue(tidx_ref, aidx_ref, text_hbm, audio_hbm,
                  rows_v.at[0], sems[0], step)
        _tc_issue(tidx_ref, aidx_ref, text_hbm, audio_hbm,
                  rows_v.at[1], sems[1], step + 1)

    for k in range(RING):
        nslot = (k + 2) % RING

        @pl.when((parity == k) & (step + 2 < NSTEP))
        def _(nslot=nslot):
            _tc_issue(tidx_ref, aidx_ref, text_hbm, audio_hbm,
                      rows_v.at[nslot], sems[nslot], step + 2)

    for k in range(RING):
        @pl.when(parity == k)
        def _(k=k):
            pltpu.make_async_copy(
                audio_hbm.at[pl.ds(0, RPS), :], rows_v.at[k], sems[k]).wait()

    for p in range(PB):
        acc = None
        for g in range(NUM_CB // 8):
            rows8 = rows_v[parity, pl.ds(p * NUM_CB + g * 8, 8), :]
            scales = jnp.stack([
                jnp.where(
                    aidx_ref[step * ARS + p * NUM_CB + g * 8 + k]
                    != (g * 8 + k) * AV3, 1.0, 0.0).astype(jnp.float32)
                for k in range(8)
            ]).reshape(8, 1)
            term = rows8 * scales
            acc = term if acc is None else acc + term
        trow = rows_v[parity, pl.ds(ARS + p, 1), :]
        tscale = jnp.where(
            tidx_ref[step * PB + p] != 0, 1.0, 0.0).astype(jnp.float32)
        result = jnp.sum(acc, axis=0, keepdims=True) + trow * tscale
        out_ref[pl.ds(p, 1), :] = result


_tc_call = pl.pallas_call(
    _tc_kernel,
    grid_spec=pltpu.PrefetchScalarGridSpec(
        num_scalar_prefetch=2,
        grid=(NSTEP,),
        in_specs=[
            pl.BlockSpec(memory_space=pl.ANY),
            pl.BlockSpec(memory_space=pl.ANY),
        ],
        out_specs=pl.BlockSpec((PB, H), lambda i, t, a: (i, 0)),
        scratch_shapes=[
            pltpu.VMEM((RING, RPS, H), jnp.float32),
            pltpu.SemaphoreType.DMA,
            pltpu.SemaphoreType.DMA,
            pltpu.SemaphoreType.DMA,
        ],
    ),
    out_shape=jax.ShapeDtypeStruct((NP_TC, H), jnp.float32),
)


def kernel(input_ids, text_table, audio_table):
    ids = input_ids.reshape(P, NUM_CB + 1).astype(jnp.int32)
    ids_pad = (
        jnp.pad(ids[:P_SC], ((0, 0), (0, IDS_PAD - (NUM_CB + 1))))
        .reshape(-1))
    sc_out = _embed_kernel(ids_pad, text_table, audio_table)

    tc_ids = ids[P_SC:]
    offs = jnp.arange(NUM_CB, dtype=jnp.int32) * AV3
    tc_aidx = (tc_ids[:, :-1] + offs[None, :]).reshape(-1)
    tc_tidx = tc_ids[:, -1].reshape(-1)
    tc_out = _tc_call(tc_tidx, tc_aidx, text_table, audio_table)

    out = lax.dynamic_update_slice(sc_out, tc_out, (P_SC, 0))
    return out.reshape(B, S, H)
